# Initial kernel scaffold; baseline (speedup 1.0000x reference)
import jax
import jax.numpy as jnp
from jax.experimental import pallas as pl


def kernel(question_embeddings, question_entities_masks, rel_emb, edge_index, edge_type, Wq, bq, Wr, br, W1, b1, W2, b2, Wo, bo):
    raise NotImplementedError("write your pallas kernel here")


# speedup vs baseline: 14.2566x; 14.2566x over previous
import functools
import jax
import jax.numpy as jnp
from jax import lax
from jax.experimental import pallas as pl
from jax.experimental.pallas import tpu as pltpu
from jax.experimental.pallas import tpu_sc as plsc

N = 10000
NPAD = 10240
D = 128
E = 160000
B = 4
CHUNK = 80
N_TILES = 16
EDGE_ROWS = E // CHUNK
ROWS_PER_TILE = EDGE_ROWS // N_TILES
NODE_ROWS_PER_TILE = NPAD // N_TILES
F32 = jnp.float32


def _proj_body(x_ref, w_ref, b_ref, o_ref):
    o_ref[...] = (
        jnp.dot(x_ref[...], w_ref[...], preferred_element_type=F32) + b_ref[...]
    )


def _rel_proj(rel_emb_pad, Wr, br2):
    return pl.pallas_call(
        _proj_body,
        out_shape=jax.ShapeDtypeStruct((rel_emb_pad.shape[0], D), F32),
    )(rel_emb_pad, Wr, br2)


def _boundary_body(mask_ref, qe_ref, wq_ref, bq_ref, wo2_ref, bo_ref,
                   bnd_ref, qt_ref):
    q = jnp.dot(qe_ref[...], wq_ref[...], preferred_element_type=F32) + bq_ref[...]
    bnd_ref[...] = mask_ref[...][:, :, None] * q[:, None, :]
    qt_ref[...] = jnp.dot(q, wo2_ref[...], preferred_element_type=F32) + bo_ref[...]


def _boundary(mask, qe, Wq, bq2, Wo2, bo2):
    bn = 1024
    return pl.pallas_call(
        _boundary_body,
        grid=(NPAD // bn,),
        in_specs=[
            pl.BlockSpec((B, bn), lambda i: (0, i)),
            pl.BlockSpec((B, qe.shape[1]), lambda i: (0, 0)),
            pl.BlockSpec(Wq.shape, lambda i: (0, 0)),
            pl.BlockSpec(bq2.shape, lambda i: (0, 0)),
            pl.BlockSpec(Wo2.shape, lambda i: (0, 0)),
            pl.BlockSpec(bo2.shape, lambda i: (0, 0)),
        ],
        out_specs=[
            pl.BlockSpec((B, bn, D), lambda i: (0, i, 0)),
            pl.BlockSpec((B, 1), lambda i: (0, 0)),
        ],
        out_shape=[
            jax.ShapeDtypeStruct((B, NPAD, D), F32),
            jax.ShapeDtypeStruct((B, 1), F32),
        ],
    )(mask, qe, Wq, bq2, Wo2, bo2)


def _update_body(a_ref, w_ref, b_ref, o_ref):
    o_ref[...] = jnp.maximum(
        jnp.dot(a_ref[...], w_ref[...], preferred_element_type=F32) + b_ref[...],
        0.0,
    )


def _update(agg, W, b2):
    bm = 1024
    return pl.pallas_call(
        _update_body,
        grid=(B * NPAD // bm,),
        in_specs=[
            pl.BlockSpec((bm, D), lambda i: (i, 0)),
            pl.BlockSpec(W.shape, lambda i: (0, 0)),
            pl.BlockSpec(b2.shape, lambda i: (0, 0)),
        ],
        out_specs=pl.BlockSpec((bm, D), lambda i: (i, 0)),
        out_shape=jax.ShapeDtypeStruct((B * NPAD, D), F32),
    )(agg, W, b2)


def _final_body(a_ref, w2_ref, b2_ref, wo1_ref, qt_ref, o_ref):
    h = jnp.maximum(
        jnp.dot(a_ref[...], w2_ref[...], preferred_element_type=F32) + b2_ref[...],
        0.0,
    )
    s = jnp.dot(h, wo1_ref[...], preferred_element_type=F32)
    bidx = pl.program_id(0) // (NPAD // 1024)
    o_ref[...] = s + qt_ref[bidx, 0]


def _final(agg2, W2, b2_, Wo1, qterm):
    bm = 1024
    return pl.pallas_call(
        _final_body,
        grid=(B * NPAD // bm,),
        in_specs=[
            pl.BlockSpec((bm, D), lambda i: (i, 0)),
            pl.BlockSpec(W2.shape, lambda i: (0, 0)),
            pl.BlockSpec(b2_.shape, lambda i: (0, 0)),
            pl.BlockSpec(Wo1.shape, lambda i: (0, 0)),
            pl.BlockSpec(memory_space=pltpu.SMEM),
        ],
        out_specs=pl.BlockSpec((bm, 1), lambda i: (i, 0)),
        out_shape=jax.ShapeDtypeStruct((B * NPAD, 1), F32),
    )(agg2, W2, b2_, Wo1, qterm)


def _sc_agg_body(hid, bnd, rel, srcm, dstm, etm, agg,
                 src_c, dst_c, et_c, gsrc_c, hrows, rrows, acc, sem0, sem1):
    cid = lax.axis_index("c")
    sid = lax.axis_index("s")
    node_base = sid * NODE_ROWS_PER_TILE

    for bi in range(2):
        b = cid + 2 * bi
        hid_base = b * NPAD
        pltpu.sync_copy(
            bnd.at[pl.ds(hid_base + node_base, NODE_ROWS_PER_TILE)],
            acc.at[pl.ds(node_base, NODE_ROWS_PER_TILE)],
        )
        plsc.subcore_barrier()

        def chunk_step(i, _):
            crow = sid * ROWS_PER_TILE + i
            pltpu.sync_copy(srcm.at[crow], src_c)
            pltpu.sync_copy(dstm.at[crow], dst_c)
            pltpu.sync_copy(etm.at[crow], et_c)
            for k in range(CHUNK // 16):
                gsrc_c[pl.ds(16 * k, 16)] = src_c[pl.ds(16 * k, 16)] + hid_base
            cp_h = pltpu.async_copy(hid.at[gsrc_c], hrows, sem0)
            cp_r = pltpu.async_copy(rel.at[et_c], rrows, sem1)
            cp_h.wait()
            cp_r.wait()

            def mul_step(j, _):
                for k in range(D // 16):
                    sl = pl.ds(16 * k, 16)
                    hrows[j, sl] = hrows[j, sl] * rrows[j, sl]
                return 0

            lax.fori_loop(0, CHUNK, mul_step, 0)
            pltpu.sync_copy(hrows, acc.at[dst_c], add=True)
            return 0

        lax.fori_loop(0, ROWS_PER_TILE, chunk_step, 0)
        plsc.subcore_barrier()

        pltpu.sync_copy(
            acc.at[pl.ds(node_base, NODE_ROWS_PER_TILE)],
            agg.at[pl.ds(hid_base + node_base, NODE_ROWS_PER_TILE)],
        )
        plsc.subcore_barrier()


@functools.partial(
    pl.kernel,
    out_type=jax.ShapeDtypeStruct((B * NPAD, D), F32),
    mesh=plsc.VectorSubcoreMesh(core_axis_name="c", subcore_axis_name="s"),
    scratch_types=[
        pltpu.VMEM((CHUNK,), jnp.int32),
        pltpu.VMEM((CHUNK,), jnp.int32),
        pltpu.VMEM((CHUNK,), jnp.int32),
        pltpu.VMEM((CHUNK,), jnp.int32),
        pltpu.VMEM((CHUNK, D), F32),
        pltpu.VMEM((CHUNK, D), F32),
        pltpu.VMEM_SHARED((NPAD, D), F32),
        pltpu.SemaphoreType.DMA,
        pltpu.SemaphoreType.DMA,
    ],
)
def _sc_agg(hid, bnd, rel, srcm, dstm, etm, agg, *scratch):
    _sc_agg_body(hid, bnd, rel, srcm, dstm, etm, agg, *scratch)


def kernel(question_embeddings, question_entities_masks, rel_emb, edge_index,
           edge_type, Wq, bq, Wr, br, W1, b1, W2, b2, Wo, bo):
    src = edge_index[0].reshape(EDGE_ROWS, CHUNK)
    dst = edge_index[1].reshape(EDGE_ROWS, CHUNK)
    et = edge_type.reshape(EDGE_ROWS, CHUNK)

    rel_emb_pad = jnp.pad(rel_emb, ((0, 512 - rel_emb.shape[0]), (0, 0)))
    rel_p = _rel_proj(rel_emb_pad, Wr, br.reshape(1, D))

    mask_pad = jnp.pad(question_entities_masks, ((0, 0), (0, NPAD - N)))
    boundary, qterm = _boundary(
        mask_pad, question_embeddings, Wq, bq.reshape(1, D),
        Wo[D:], bo.reshape(1, 1),
    )
    bflat = boundary.reshape(B * NPAD, D)

    agg1 = _sc_agg(bflat, bflat, rel_p, src, dst, et)
    h1 = _update(agg1, W1, b1.reshape(1, D))
    agg2 = _sc_agg(h1, bflat, rel_p, src, dst, et)
    out = _final(agg2, W2, b2.reshape(1, D), Wo[:D], qterm)
    return out.reshape(B, NPAD)[:, :N]


# speedup vs baseline: 29.3529x; 2.0589x over previous
import functools
import jax
import jax.numpy as jnp
from jax import lax
from jax.experimental import pallas as pl
from jax.experimental.pallas import tpu as pltpu
from jax.experimental.pallas import tpu_sc as plsc

N = 10000
NPAD = 10240
D = 128
E = 160000
B = 4
CHUNK = 80
N_TILES = 16
EDGES_PER_TILE = E // N_TILES
CHUNKS_PER_TILE = EDGES_PER_TILE // CHUNK
NODE_ROWS_PER_TILE = NPAD // N_TILES
F32 = jnp.float32


def _proj_body(x_ref, w_ref, b_ref, o_ref):
    o_ref[...] = (
        jnp.dot(x_ref[...], w_ref[...], preferred_element_type=F32) + b_ref[...]
    )


def _rel_proj(rel_emb_pad, Wr, br2):
    return pl.pallas_call(
        _proj_body,
        out_shape=jax.ShapeDtypeStruct((rel_emb_pad.shape[0], D), F32),
    )(rel_emb_pad, Wr, br2)


def _boundary_body(mask_ref, qe_ref, wq_ref, bq_ref, wo2_ref, bo_ref,
                   bnd_ref, qt_ref):
    q = jnp.dot(qe_ref[...], wq_ref[...], preferred_element_type=F32) + bq_ref[...]
    bnd_ref[...] = mask_ref[...][:, :, None] * q[:, None, :]
    qt_ref[...] = jnp.dot(q, wo2_ref[...], preferred_element_type=F32) + bo_ref[...]


def _boundary(mask, qe, Wq, bq2, Wo2, bo2):
    bn = 1024
    return pl.pallas_call(
        _boundary_body,
        grid=(NPAD // bn,),
        in_specs=[
            pl.BlockSpec((B, bn), lambda i: (0, i)),
            pl.BlockSpec((B, qe.shape[1]), lambda i: (0, 0)),
            pl.BlockSpec(Wq.shape, lambda i: (0, 0)),
            pl.BlockSpec(bq2.shape, lambda i: (0, 0)),
            pl.BlockSpec(Wo2.shape, lambda i: (0, 0)),
            pl.BlockSpec(bo2.shape, lambda i: (0, 0)),
        ],
        out_specs=[
            pl.BlockSpec((B, bn, D), lambda i: (0, i, 0)),
            pl.BlockSpec((B, 1), lambda i: (0, 0)),
        ],
        out_shape=[
            jax.ShapeDtypeStruct((B, NPAD, D), F32),
            jax.ShapeDtypeStruct((B, 1), F32),
        ],
    )(mask, qe, Wq, bq2, Wo2, bo2)


def _update_body(a_ref, w_ref, b_ref, o_ref):
    o_ref[...] = jnp.maximum(
        jnp.dot(a_ref[...], w_ref[...], preferred_element_type=F32) + b_ref[...],
        0.0,
    )


def _update(agg, W, b2):
    bm = 1024
    return pl.pallas_call(
        _update_body,
        grid=(B * NPAD // bm,),
        in_specs=[
            pl.BlockSpec((bm, D), lambda i: (i, 0)),
            pl.BlockSpec(W.shape, lambda i: (0, 0)),
            pl.BlockSpec(b2.shape, lambda i: (0, 0)),
        ],
        out_specs=pl.BlockSpec((bm, D), lambda i: (i, 0)),
        out_shape=jax.ShapeDtypeStruct((B * NPAD, D), F32),
    )(agg, W, b2)


def _final_body(a_ref, w2_ref, b2_ref, wo1_ref, qt_ref, o_ref):
    h = jnp.maximum(
        jnp.dot(a_ref[...], w2_ref[...], preferred_element_type=F32) + b2_ref[...],
        0.0,
    )
    s = jnp.dot(h, wo1_ref[...], preferred_element_type=F32)
    bidx = pl.program_id(0) // (NPAD // 1024)
    o_ref[...] = s + qt_ref[bidx, 0]


def _final(agg2, W2, b2_, Wo1, qterm):
    bm = 1024
    return pl.pallas_call(
        _final_body,
        grid=(B * NPAD // bm,),
        in_specs=[
            pl.BlockSpec((bm, D), lambda i: (i, 0)),
            pl.BlockSpec(W2.shape, lambda i: (0, 0)),
            pl.BlockSpec(b2_.shape, lambda i: (0, 0)),
            pl.BlockSpec(Wo1.shape, lambda i: (0, 0)),
            pl.BlockSpec(memory_space=pltpu.SMEM),
        ],
        out_specs=pl.BlockSpec((bm, 1), lambda i: (i, 0)),
        out_shape=jax.ShapeDtypeStruct((B * NPAD, 1), F32),
    )(agg2, W2, b2_, Wo1, qterm)


def _sc_agg_body(hid, bnd, rel, edg, agg,
                 ib0, ib1, h0, h1, r0, r1, acc,
                 si0, si1, sh0, sh1, sr0, sr1):
    cid = lax.axis_index("c")
    sid = lax.axis_index("s")
    node_base = sid * NODE_ROWS_PER_TILE
    crow0 = sid * CHUNKS_PER_TILE

    def load_idx(ib, c, sem):
        return pltpu.async_copy(edg.at[crow0 + c], ib.at[pl.ds(0, 3)], sem)

    def wait_idx(ib, sem):
        pltpu.make_async_copy(edg.at[crow0], ib.at[pl.ds(0, 3)], sem).wait()

    def issue_data(ib, hb, rb, hs, rs):
        pltpu.async_copy(hid.at[ib.at[3]], hb, hs)
        pltpu.async_copy(rel.at[ib.at[2]], rb, rs)

    def consume(ib, hb, rb, hs, rs):
        pltpu.make_async_copy(hid.at[ib.at[3]], hb, hs).wait()
        pltpu.make_async_copy(rel.at[ib.at[2]], rb, rs).wait()

        def mul_step(j, _):
            for k in range(D // 16):
                sl = pl.ds(16 * k, 16)
                hb[j, sl] = hb[j, sl] * rb[j, sl]
            return 0

        lax.fori_loop(0, CHUNK, mul_step, 0)
        pltpu.sync_copy(hb, acc.at[ib.at[1]], add=True)

    for bi in range(2):
        b = cid + 2 * bi
        hid_base = b * NPAD

        def gsrc(ib):
            for k in range(CHUNK // 16):
                sl = pl.ds(16 * k, 16)
                ib[3, sl] = ib[0, sl] + hid_base

        pltpu.sync_copy(
            bnd.at[pl.ds(hid_base + node_base, NODE_ROWS_PER_TILE)],
            acc.at[pl.ds(node_base, NODE_ROWS_PER_TILE)],
        )
        plsc.subcore_barrier()

        load_idx(ib0, 0, si0).wait()
        gsrc(ib0)
        issue_data(ib0, h0, r0, sh0, sr0)
        load_idx(ib1, 1, si1)

        def pair_step(t, _):
            wait_idx(ib1, si1)
            gsrc(ib1)
            issue_data(ib1, h1, r1, sh1, sr1)
            consume(ib0, h0, r0, sh0, sr0)
            load_idx(ib0, 2 * t + 2, si0)
            wait_idx(ib0, si0)
            gsrc(ib0)
            issue_data(ib0, h0, r0, sh0, sr0)
            consume(ib1, h1, r1, sh1, sr1)
            load_idx(ib1, 2 * t + 3, si1)
            return 0

        lax.fori_loop(0, (CHUNKS_PER_TILE - 3) // 2, pair_step, 0)

        wait_idx(ib1, si1)
        gsrc(ib1)
        issue_data(ib1, h1, r1, sh1, sr1)
        consume(ib0, h0, r0, sh0, sr0)
        load_idx(ib0, CHUNKS_PER_TILE - 1, si0)
        wait_idx(ib0, si0)
        gsrc(ib0)
        issue_data(ib0, h0, r0, sh0, sr0)
        consume(ib1, h1, r1, sh1, sr1)
        consume(ib0, h0, r0, sh0, sr0)
        plsc.subcore_barrier()

        pltpu.sync_copy(
            acc.at[pl.ds(node_base, NODE_ROWS_PER_TILE)],
            agg.at[pl.ds(hid_base + node_base, NODE_ROWS_PER_TILE)],
        )
        plsc.subcore_barrier()


@functools.partial(
    pl.kernel,
    out_type=jax.ShapeDtypeStruct((B * NPAD, D), F32),
    mesh=plsc.VectorSubcoreMesh(core_axis_name="c", subcore_axis_name="s"),
    scratch_types=[
        pltpu.VMEM((4, CHUNK), jnp.int32),
        pltpu.VMEM((4, CHUNK), jnp.int32),
        pltpu.VMEM((CHUNK, D), F32),
        pltpu.VMEM((CHUNK, D), F32),
        pltpu.VMEM((CHUNK, D), F32),
        pltpu.VMEM((CHUNK, D), F32),
        pltpu.VMEM_SHARED((NPAD, D), F32),
        pltpu.SemaphoreType.DMA,
        pltpu.SemaphoreType.DMA,
        pltpu.SemaphoreType.DMA,
        pltpu.SemaphoreType.DMA,
        pltpu.SemaphoreType.DMA,
        pltpu.SemaphoreType.DMA,
    ],
)
def _sc_agg(hid, bnd, rel, edg, agg, *scratch):
    _sc_agg_body(hid, bnd, rel, edg, agg, *scratch)


def kernel(question_embeddings, question_entities_masks, rel_emb, edge_index,
           edge_type, Wq, bq, Wr, br, W1, b1, W2, b2, Wo, bo):
    src = edge_index[0].reshape(N_TILES * CHUNKS_PER_TILE, CHUNK)
    dst = edge_index[1].reshape(N_TILES * CHUNKS_PER_TILE, CHUNK)
    et = edge_type.reshape(N_TILES * CHUNKS_PER_TILE, CHUNK)
    edg = jnp.stack([src, dst, et], axis=1)

    rel_emb_pad = jnp.pad(rel_emb, ((0, 512 - rel_emb.shape[0]), (0, 0)))
    rel_p = _rel_proj(rel_emb_pad, Wr, br.reshape(1, D))

    mask_pad = jnp.pad(question_entities_masks, ((0, 0), (0, NPAD - N)))
    boundary, qterm = _boundary(
        mask_pad, question_embeddings, Wq, bq.reshape(1, D),
        Wo[D:], bo.reshape(1, 1),
    )
    bflat = boundary.reshape(B * NPAD, D)

    agg1 = _sc_agg(bflat, bflat, rel_p, edg)
    h1 = _update(agg1, W1, b1.reshape(1, D))
    agg2 = _sc_agg(h1, bflat, rel_p, edg)
    out = _final(agg2, W2, b2.reshape(1, D), Wo[:D], qterm)
    return out.reshape(B, NPAD)[:, :N]
